# baseline (speedup 1.0000x reference)
import jax
import jax.numpy as jnp
from jax.experimental import pallas as pl
from jax.experimental.pallas import tpu as pltpu

_D0, _D1, _K = 2048, 2048, 4
_Q, _C = 64, 128
_BI = 256
_NI = _D0 // _BI


def _mask_kernel(x_ref, o_ref):
    b = pl.program_id(0)
    rows = jax.lax.broadcasted_iota(jnp.int32, (_BI, _Q, _C), 0) + b * _BI
    qs = jax.lax.broadcasted_iota(jnp.int32, (_BI, _Q, _C), 1)
    cs = jax.lax.broadcasted_iota(jnp.int32, (_BI, _Q, _C), 2)
    keep = qs * _C + cs >= 4 * rows
    o_ref[...] = jnp.where(keep, x_ref[...], 0.0)


def kernel(w):
    x = w.reshape(_D0, _Q, _C)
    out = pl.pallas_call(
        _mask_kernel,
        grid=(_NI,),
        in_specs=[pl.BlockSpec((_BI, _Q, _C), lambda b: (b, 0, 0))],
        out_specs=pl.BlockSpec((_BI, _Q, _C), lambda b: (b, 0, 0)),
        out_shape=jax.ShapeDtypeStruct((_D0, _Q, _C), jnp.float32),
    )(x)
    return out.reshape(w.shape)


# speedup vs baseline: 10.4241x; 10.4241x over previous
import jax
import jax.numpy as jnp
from jax.experimental import pallas as pl
from jax.experimental.pallas import tpu as pltpu

_D0, _D1, _K = 2048, 2048, 4
_Q, _C = 64, 128
_BI = 256
_NI = _D0 // _BI


def _mask_kernel(x_ref, o_ref):
    b = pl.program_id(0)
    rows = jax.lax.broadcasted_iota(jnp.int32, (_BI, _Q, _C), 0) + b * _BI
    qs = jax.lax.broadcasted_iota(jnp.int32, (_BI, _Q, _C), 1)
    cs = jax.lax.broadcasted_iota(jnp.int32, (_BI, _Q, _C), 2)
    keep = (qs >> 2) * _C + cs >= rows
    o_ref[...] = jnp.where(keep, x_ref[...], 0.0)


def kernel(w):
    x = (w.reshape(_D0, 16, _C, _K)
          .transpose(0, 1, 3, 2)
          .reshape(_D0, _Q, _C))
    out = pl.pallas_call(
        _mask_kernel,
        grid=(_NI,),
        in_specs=[pl.BlockSpec((_BI, _Q, _C), lambda b: (b, 0, 0))],
        out_specs=pl.BlockSpec((_BI, _Q, _C), lambda b: (b, 0, 0)),
        out_shape=jax.ShapeDtypeStruct((_D0, _Q, _C), jnp.float32),
    )(x)
    return (out.reshape(_D0, 16, _K, _C)
               .transpose(0, 1, 3, 2)
               .reshape(_D0, _D1, _K))
